# baseline (speedup 1.0000x reference)
import functools

import jax
import jax.numpy as jnp
from jax import lax
from jax.experimental import pallas as pl
from jax.experimental.pallas import tpu as pltpu
from jax.experimental.pallas import tpu_sc as plsc

_DIMS = 32
_NUM_WORKERS = 32
_TC = 800


def _table_transpose(num_rows):
    n_chunks = num_rows // _TC
    n_slots = -(-n_chunks // _NUM_WORKERS)
    n_pairs = -(-n_slots // 2)
    mesh = plsc.VectorSubcoreMesh(core_axis_name="c", subcore_axis_name="s")

    scratch = (
        [pltpu.VMEM((_DIMS, _TC), jnp.float32) for _ in range(2)]
        + [pltpu.VMEM((_TC, _DIMS), jnp.float32) for _ in range(2)]
        + [pltpu.SemaphoreType.DMA for _ in range(4)]
    )

    @functools.partial(
        pl.kernel,
        mesh=mesh,
        out_type=jax.ShapeDtypeStruct((num_rows, _DIMS), jnp.float32),
        scratch_types=scratch,
        compiler_params=pltpu.CompilerParams(
            use_tc_tiling_on_sc=False, needs_layout_passes=False
        ),
    )
    def k(src_hbm, dst_hbm, i0, i1, t0, t1, g0, g1, o0, o1):
        tin = (i0, i1)
        tout = (t0, t1)
        g_sem = (g0, g1)
        o_sem = (o0, o1)

        wid = lax.axis_index("s") * 2 + lax.axis_index("c")
        lane = jnp.arange(16, dtype=jnp.int32)

        def transpose(b):
            src = tin[b]
            dst = tout[b]

            def body(bg, carry):
                cols = bg * 16 + lane
                for d0 in range(_DIMS):
                    dims = (d0 + lane) & (_DIMS - 1)
                    v = plsc.load_gather(src, [dims, cols])
                    plsc.store_scatter(dst, [cols, dims], v)
                return carry

            lax.fori_loop(0, _TC // 16, body, 0)

        def chunk_stage(p, b, started):
            cid = (2 * p + b) * _NUM_WORKERS + wid

            @pl.when(cid < n_chunks)
            def _():
                c0 = cid * _TC
                pltpu.async_copy(src_hbm.at[:, pl.ds(c0, _TC)], tin[b], g_sem[b])

            return cid

        def chunk_finish(p, b, cid):
            @pl.when(cid < n_chunks)
            def _():
                c0 = cid * _TC
                pltpu.make_async_copy(
                    src_hbm.at[:, pl.ds(c0, _TC)], tin[b], g_sem[b]
                ).wait()

                @pl.when(p > 0)
                def _():
                    prev0 = ((2 * (p - 1) + b) * _NUM_WORKERS + wid) * _TC
                    pltpu.make_async_copy(
                        tout[b], dst_hbm.at[pl.ds(prev0, _TC), :], o_sem[b]
                    ).wait()

                transpose(b)
                pltpu.async_copy(tout[b], dst_hbm.at[pl.ds(c0, _TC), :], o_sem[b])

        def pair(p, carry):
            c0 = chunk_stage(p, 0, None)
            c1 = chunk_stage(p, 1, None)
            chunk_finish(p, 0, c0)
            chunk_finish(p, 1, c1)
            return carry

        lax.fori_loop(0, n_pairs, pair, 0)

        for b in range(2):
            last = (2 * (n_pairs - 1) + b) * _NUM_WORKERS + wid

            @pl.when(last < n_chunks)
            def _():
                pltpu.make_async_copy(
                    tout[b], dst_hbm.at[pl.ds(last * _TC, _TC), :], o_sem[b]
                ).wait()

    return k


def _embedding_gather(batch, hist):
    b_per_w = batch // _NUM_WORKERS
    n_bg = b_per_w // 16
    mesh = plsc.VectorSubcoreMesh(core_axis_name="c", subcore_axis_name="s")

    scratch = (
        [pltpu.VMEM((hist, b_per_w), jnp.int32)]
        + [pltpu.VMEM((b_per_w, _DIMS), jnp.float32) for _ in range(2)]
        + [pltpu.VMEM((_DIMS, b_per_w), jnp.float32) for _ in range(2)]
        + [pltpu.SemaphoreType.DMA for _ in range(4)]
    )

    @functools.partial(
        pl.kernel,
        mesh=mesh,
        out_type=jax.ShapeDtypeStruct((hist, _DIMS, batch), jnp.float32),
        scratch_types=scratch,
        compiler_params=pltpu.CompilerParams(
            use_tc_tiling_on_sc=False, needs_layout_passes=False
        ),
    )
    def k(idx_hbm, table_hbm, out_hbm, idx_v, r0, r1, t0, t1, g0, g1, o0, o1):
        rows = (r0, r1)
        outb = (t0, t1)
        g_sem = (g0, g1)
        o_sem = (o0, o1)

        wid = lax.axis_index("s") * 2 + lax.axis_index("c")
        b0 = wid * b_per_w

        pltpu.sync_copy(idx_hbm.at[:, pl.ds(b0, b_per_w)], idx_v)

        lane = jnp.arange(16, dtype=jnp.int32)

        def start_gather(h):
            b = h % 2
            return pltpu.async_copy(table_hbm.at[idx_v.at[h]], rows[b], g_sem[b])

        def start_out(h):
            b = h % 2
            return pltpu.async_copy(
                outb[b], out_hbm.at[h, :, pl.ds(b0, b_per_w)], o_sem[b]
            )

        def transpose(b):
            src = rows[b]
            dst = outb[b]

            def body(bg, carry):
                bids = bg * 16 + lane
                for d0 in range(_DIMS):
                    cids = (d0 + lane) & (_DIMS - 1)
                    v = plsc.load_gather(src, [bids, cids])
                    plsc.store_scatter(dst, [cids, bids], v)
                return carry

            lax.fori_loop(0, n_bg, body, 0)

        gathers = [None] * hist
        outs = [None] * hist
        gathers[0] = start_gather(0)
        for h in range(hist):
            if h + 1 < hist:
                gathers[h + 1] = start_gather(h + 1)
            gathers[h].wait()
            if h >= 2:
                outs[h - 2].wait()
            transpose(h % 2)
            outs[h] = start_out(h)
        outs[hist - 2].wait()
        outs[hist - 1].wait()

    return k


def kernel(inputs, weight):
    batch, hist = inputs.shape
    wt_rm = _table_transpose(weight.shape[0])(weight.T)
    out_t = _embedding_gather(batch, hist)(inputs.T, wt_rm)
    return out_t.transpose(2, 0, 1)


# speedup vs baseline: 4.6614x; 4.6614x over previous
import functools

import jax
import jax.numpy as jnp
from jax import lax
from jax.experimental import pallas as pl
from jax.experimental.pallas import tpu as pltpu
from jax.experimental.pallas import tpu_sc as plsc

_DIMS = 32
_NUM_WORKERS = 32
_TC = 800


def _table_transpose(num_rows):
    n_chunks = num_rows // _TC
    n_slots = -(-n_chunks // _NUM_WORKERS)
    n_pairs = -(-n_slots // 2)
    mesh = plsc.VectorSubcoreMesh(core_axis_name="c", subcore_axis_name="s")

    scratch = (
        [pltpu.VMEM((_DIMS, _TC), jnp.float32) for _ in range(2)]
        + [pltpu.VMEM((_TC, _DIMS), jnp.float32) for _ in range(2)]
        + [pltpu.SemaphoreType.DMA for _ in range(4)]
    )

    @functools.partial(
        pl.kernel,
        mesh=mesh,
        out_type=jax.ShapeDtypeStruct((num_rows, _DIMS), jnp.float32),
        scratch_types=scratch,
        compiler_params=pltpu.CompilerParams(
            use_tc_tiling_on_sc=False, needs_layout_passes=False
        ),
    )
    def k(src_hbm, dst_hbm, i0, i1, t0, t1, g0, g1, o0, o1):
        tin = (i0, i1)
        tout = (t0, t1)
        g_sem = (g0, g1)
        o_sem = (o0, o1)

        wid = lax.axis_index("s") * 2 + lax.axis_index("c")
        lane = jnp.arange(16, dtype=jnp.int32)

        def transpose(b):
            src = tin[b]
            dst = tout[b]

            def body(bg, carry):
                cols = bg * 16 + lane
                for d0 in range(_DIMS):
                    dims = (d0 + lane) & (_DIMS - 1)
                    v = plsc.load_gather(src, [dims, cols])
                    plsc.store_scatter(dst, [cols, dims], v)
                return carry

            lax.fori_loop(0, _TC // 16, body, 0)

        def chunk_stage(p, b, started):
            cid = (2 * p + b) * _NUM_WORKERS + wid

            @pl.when(cid < n_chunks)
            def _():
                c0 = cid * _TC
                pltpu.async_copy(src_hbm.at[:, pl.ds(c0, _TC)], tin[b], g_sem[b])

            return cid

        def chunk_finish(p, b, cid):
            @pl.when(cid < n_chunks)
            def _():
                c0 = cid * _TC
                pltpu.make_async_copy(
                    src_hbm.at[:, pl.ds(c0, _TC)], tin[b], g_sem[b]
                ).wait()

                @pl.when(p > 0)
                def _():
                    prev0 = ((2 * (p - 1) + b) * _NUM_WORKERS + wid) * _TC
                    pltpu.make_async_copy(
                        tout[b], dst_hbm.at[pl.ds(prev0, _TC), :], o_sem[b]
                    ).wait()

                transpose(b)
                pltpu.async_copy(tout[b], dst_hbm.at[pl.ds(c0, _TC), :], o_sem[b])

        def pair(p, carry):
            c0 = chunk_stage(p, 0, None)
            c1 = chunk_stage(p, 1, None)
            chunk_finish(p, 0, c0)
            chunk_finish(p, 1, c1)
            return carry

        lax.fori_loop(0, n_pairs, pair, 0)

        for b in range(2):
            last = (2 * (n_pairs - 1) + b) * _NUM_WORKERS + wid

            @pl.when(last < n_chunks)
            def _():
                pltpu.make_async_copy(
                    tout[b], dst_hbm.at[pl.ds(last * _TC, _TC), :], o_sem[b]
                ).wait()

    return k


def _embedding_gather(batch, hist):
    b_per_w = batch // _NUM_WORKERS
    n_bg = b_per_w // 16
    mesh = plsc.VectorSubcoreMesh(core_axis_name="c", subcore_axis_name="s")

    scratch = (
        [pltpu.VMEM((hist, b_per_w), jnp.int32)]
        + [pltpu.VMEM((b_per_w, _DIMS), jnp.float32) for _ in range(2)]
        + [pltpu.VMEM((_DIMS, b_per_w), jnp.float32) for _ in range(2)]
        + [pltpu.SemaphoreType.DMA for _ in range(4)]
    )

    @functools.partial(
        pl.kernel,
        mesh=mesh,
        out_type=jax.ShapeDtypeStruct((hist, _DIMS, batch), jnp.float32),
        scratch_types=scratch,
        compiler_params=pltpu.CompilerParams(
            use_tc_tiling_on_sc=False, needs_layout_passes=False
        ),
    )
    def k(idx_hbm, table_hbm, out_hbm, idx_v, r0, r1, t0, t1, g0, g1, o0, o1):
        rows = (r0, r1)
        outb = (t0, t1)
        g_sem = (g0, g1)
        o_sem = (o0, o1)

        wid = lax.axis_index("s") * 2 + lax.axis_index("c")
        b0 = wid * b_per_w

        pltpu.sync_copy(idx_hbm.at[:, pl.ds(b0, b_per_w)], idx_v)

        lane = jnp.arange(16, dtype=jnp.int32)

        def start_gather(h):
            b = h % 2
            return pltpu.async_copy(table_hbm.at[idx_v.at[h]], rows[b], g_sem[b])

        def start_out(h):
            b = h % 2
            return pltpu.async_copy(
                outb[b], out_hbm.at[h, :, pl.ds(b0, b_per_w)], o_sem[b]
            )

        def transpose(b):
            src = rows[b]
            dst = outb[b]

            def body(bg, carry):
                bids = bg * 16 + lane
                for d0 in range(_DIMS):
                    cids = (d0 + lane) & (_DIMS - 1)
                    v = plsc.load_gather(src, [bids, cids])
                    plsc.store_scatter(dst, [cids, bids], v)
                return carry

            lax.fori_loop(0, n_bg, body, 0)

        gathers = [None] * hist
        outs = [None] * hist
        gathers[0] = start_gather(0)
        for h in range(hist):
            if h + 1 < hist:
                gathers[h + 1] = start_gather(h + 1)
            gathers[h].wait()
            if h >= 2:
                outs[h - 2].wait()
            transpose(h % 2)
            outs[h] = start_out(h)
        outs[hist - 2].wait()
        outs[hist - 1].wait()

    return k


def kernel(inputs, weight):
    batch, hist = inputs.shape
    out_t = _embedding_gather(batch, hist)(inputs.T, weight)
    return out_t.transpose(2, 0, 1)
